# Initial kernel scaffold; baseline (speedup 1.0000x reference)
import jax
import jax.numpy as jnp
from jax.experimental import pallas as pl


def kernel(X, edge_index, W1, W2):
    raise NotImplementedError("write your pallas kernel here")


# speedup vs baseline: 29.2148x; 29.2148x over previous
import functools

import jax
import jax.numpy as jnp
from jax import lax
from jax.experimental import pallas as pl
from jax.experimental.pallas import tpu as pltpu
from jax.experimental.pallas import tpu_sc as plsc

N_NODES = 10000
N_EDGES = 320000
D = 128

_info = plsc.get_sparse_core_info()
_NC, _NS, _L = _info.num_cores, _info.num_subcores, _info.num_lanes
_NW = _NC * _NS
_EPW = N_EDGES // _NW
_CHUNKS = _EPW // _L


def _proj_body(x_ref, w_ref, o_ref):
    o_ref[...] = jnp.dot(x_ref[...], w_ref[...],
                         preferred_element_type=jnp.float32)


def _project(X, Wc):
    return pl.pallas_call(
        _proj_body,
        out_shape=jax.ShapeDtypeStruct((N_NODES, 2), jnp.float32),
    )(X, Wc)


@functools.partial(
    pl.kernel,
    out_type=jax.ShapeDtypeStruct((N_EDGES,), jnp.float32),
    mesh=plsc.VectorSubcoreMesh(core_axis_name="c", subcore_axis_name="s"),
    compiler_params=pltpu.CompilerParams(needs_layout_passes=False),
    scratch_types=[
        pltpu.VMEM((2 * N_NODES,), jnp.float32),
        pltpu.VMEM((_EPW,), jnp.int32),
        pltpu.VMEM((_EPW,), jnp.int32),
        pltpu.VMEM((_EPW,), jnp.float32),
    ],
)
def _sc_gather_add(y_hbm, src_hbm, dst_hbm, out_hbm, y_v, src_v, dst_v, out_v):
    wid = lax.axis_index("s") * _NC + lax.axis_index("c")
    base = wid * _EPW
    pltpu.sync_copy(y_hbm, y_v)
    pltpu.sync_copy(src_hbm.at[pl.ds(base, _EPW)], src_v)
    pltpu.sync_copy(dst_hbm.at[pl.ds(base, _EPW)], dst_v)

    def body(i, carry):
        off = i * _L
        s = src_v[pl.ds(off, _L)]
        d = dst_v[pl.ds(off, _L)]
        a = plsc.load_gather(y_v, [s * 2])
        b = plsc.load_gather(y_v, [d * 2 + 1])
        out_v[pl.ds(off, _L)] = a + b
        return carry

    lax.fori_loop(0, _CHUNKS, body, None)
    pltpu.sync_copy(out_v, out_hbm.at[pl.ds(base, _EPW)])


def kernel(X, edge_index, W1, W2):
    Wc = jnp.concatenate([W1, W2], axis=1)
    y = _project(X, Wc)
    out = _sc_gather_add(y.reshape(-1), edge_index[0], edge_index[1])
    return out[:, None]


# speedup vs baseline: 30.9124x; 1.0581x over previous
import functools

import jax
import jax.numpy as jnp
from jax import lax
from jax.experimental import pallas as pl
from jax.experimental.pallas import tpu as pltpu
from jax.experimental.pallas import tpu_sc as plsc

N_NODES = 10000
N_EDGES = 320000
D = 128

_info = plsc.get_sparse_core_info()
_NC, _NS, _L = _info.num_cores, _info.num_subcores, _info.num_lanes
_NW = _NC * _NS
_EPW = N_EDGES // _NW
_CHUNKS = _EPW // _L


def _proj_body(x_ref, w_ref, o_ref):
    o_ref[...] = jnp.dot(x_ref[...], w_ref[...],
                         preferred_element_type=jnp.float32)


def _project(X, Wc):
    return pl.pallas_call(
        _proj_body,
        out_shape=jax.ShapeDtypeStruct((N_NODES, 2), jnp.float32),
    )(X, Wc)


@functools.partial(
    pl.kernel,
    out_type=jax.ShapeDtypeStruct((N_EDGES,), jnp.float32),
    mesh=plsc.VectorSubcoreMesh(core_axis_name="c", subcore_axis_name="s"),
    compiler_params=pltpu.CompilerParams(needs_layout_passes=False),
    scratch_types=[
        pltpu.VMEM((2 * N_NODES,), jnp.float32),
        pltpu.VMEM((_EPW,), jnp.int32),
        pltpu.VMEM((_EPW,), jnp.int32),
        pltpu.VMEM((_EPW,), jnp.float32),
        pltpu.SemaphoreType.DMA,
    ],
)
def _sc_gather_add(y_hbm, src_hbm, dst_hbm, out_hbm, y_v, src_v, dst_v, out_v,
                   sem):
    wid = lax.axis_index("s") * _NC + lax.axis_index("c")
    base = wid * _EPW
    c1 = pltpu.async_copy(y_hbm, y_v, sem)
    c2 = pltpu.async_copy(src_hbm.at[pl.ds(base, _EPW)], src_v, sem)
    c3 = pltpu.async_copy(dst_hbm.at[pl.ds(base, _EPW)], dst_v, sem)
    c1.wait()
    c2.wait()
    c3.wait()

    @plsc.parallel_loop(0, _EPW, step=_L, unroll=8)
    def _body(off):
        s = src_v[pl.ds(off, _L)]
        d = dst_v[pl.ds(off, _L)]
        a = plsc.load_gather(y_v, [s * 2])
        b = plsc.load_gather(y_v, [d * 2 + 1])
        out_v[pl.ds(off, _L)] = a + b

    pltpu.sync_copy(out_v, out_hbm.at[pl.ds(base, _EPW)])


def kernel(X, edge_index, W1, W2):
    Wc = jnp.concatenate([W1, W2], axis=1)
    y = _project(X, Wc)
    out = _sc_gather_add(y.reshape(-1), edge_index[0], edge_index[1])
    return out[:, None]


# speedup vs baseline: 40.0027x; 1.2941x over previous
import functools

import jax
import jax.numpy as jnp
from jax import lax
from jax.experimental import pallas as pl
from jax.experimental.pallas import tpu as pltpu
from jax.experimental.pallas import tpu_sc as plsc

N_NODES = 10000
N_EDGES = 320000
D = 128

_info = plsc.get_sparse_core_info()
_NC, _NS, _L = _info.num_cores, _info.num_subcores, _info.num_lanes
_NW = _NC * _NS
_EPW = N_EDGES // _NW
_CHUNKS = _EPW // _L


def _proj_body(x_ref, w_ref, o_ref):
    o_ref[...] = jnp.dot(x_ref[...], w_ref[...],
                         preferred_element_type=jnp.float32)


def _project(X, Wc):
    return pl.pallas_call(
        _proj_body,
        out_shape=jax.ShapeDtypeStruct((N_NODES, 2), jnp.float32),
    )(X, Wc)


@functools.partial(
    pl.kernel,
    out_type=jax.ShapeDtypeStruct((N_EDGES,), jnp.float32),
    mesh=plsc.VectorSubcoreMesh(core_axis_name="c", subcore_axis_name="s"),
    compiler_params=pltpu.CompilerParams(needs_layout_passes=False),
    scratch_types=[
        pltpu.VMEM((_EPW,), jnp.float32),
        pltpu.SemaphoreType.DMA,
    ],
)
def _sc_floor_probe(src_hbm, out_hbm, out_v, sem):
    wid = lax.axis_index("s") * _NC + lax.axis_index("c")
    base = wid * _EPW
    pltpu.sync_copy(src_hbm.at[pl.ds(base, _EPW)], out_v)
    pltpu.sync_copy(out_v, out_hbm.at[pl.ds(base, _EPW)])


@functools.partial(
    pl.kernel,
    out_type=jax.ShapeDtypeStruct((N_EDGES,), jnp.float32),
    mesh=plsc.VectorSubcoreMesh(core_axis_name="c", subcore_axis_name="s"),
    compiler_params=pltpu.CompilerParams(needs_layout_passes=False),
    scratch_types=[
        pltpu.VMEM((2 * N_NODES,), jnp.float32),
        pltpu.VMEM((_EPW,), jnp.int32),
        pltpu.VMEM((_EPW,), jnp.int32),
        pltpu.VMEM((_EPW,), jnp.float32),
        pltpu.VMEM_SHARED((2 * N_NODES,), jnp.float32),
        pltpu.SemaphoreType.DMA,
    ],
)
def _sc_gather_add(y_hbm, edge_hbm, out_hbm, y_v, src_v, dst_v, out_v,
                   y_sp, sem):
    wid = lax.axis_index("s") * _NC + lax.axis_index("c")
    base = wid * _EPW
    c2 = pltpu.async_copy(edge_hbm.at[pl.ds(base, _EPW)], src_v, sem)
    c3 = pltpu.async_copy(edge_hbm.at[pl.ds(N_EDGES + base, _EPW)], dst_v, sem)
    @pl.when(lax.axis_index("s") == 0)
    def _():
        pltpu.sync_copy(y_hbm, y_sp)

    plsc.subcore_barrier()
    pltpu.sync_copy(y_sp, y_v)
    c2.wait()
    c3.wait()

    @plsc.parallel_loop(0, _EPW, step=_L, unroll=8)
    def _body(off):
        s = src_v[pl.ds(off, _L)]
        d = dst_v[pl.ds(off, _L)]
        a = plsc.load_gather(y_v, [s * 2])
        b = plsc.load_gather(y_v, [d * 2 + 1])
        out_v[pl.ds(off, _L)] = a + b

    pltpu.sync_copy(out_v, out_hbm.at[pl.ds(base, _EPW)])


def kernel(X, edge_index, W1, W2):
    Wc = jnp.concatenate([W1, W2], axis=1)
    y = _project(X, Wc)
    out = _sc_gather_add(y.reshape(-1), edge_index.reshape(-1))
    return out[:, None]
